# baseline (speedup 1.0000x reference)
import functools

import jax
import jax.numpy as jnp
from jax import lax
from jax.experimental import pallas as pl
from jax.experimental.pallas import tpu as pltpu
from jax.experimental.pallas import tpu_sc as plsc

N = 10000
DEG = 16
D = 256
DW = D // 2
LANES = 16

NW = 32
NPW = 320
CHUNK = 8
NCHUNKS = NPW // CHUNK
IPC = CHUNK * DEG
E_PER_W = NPW * DEG
E_TAIL = (N - (NW - 1) * NPW) * DEG

MM_BLOCK = 1000

_RND = 0x8000
_HI = -65536


def _mm_body(f_ref, w_ref, b_ref, o_ref):
    acc = lax.dot_general(
        f_ref[...], w_ref[...],
        dimension_numbers=(((1,), (1,)), ((), ())),
        preferred_element_type=jnp.float32,
    )
    r = jnp.maximum(acc + b_ref[...], 0.0) * (1.0 / DEG)
    lo = lax.bitcast_convert_type(r[:, :DW], jnp.int32)
    hi = lax.bitcast_convert_type(r[:, DW:], jnp.int32)
    o_ref[...] = lax.shift_right_logical(lo + _RND, 16) | ((hi + _RND) & _HI)


def _linear_relu_packed(feats, W, b_row):
    return pl.pallas_call(
        _mm_body,
        grid=(N // MM_BLOCK,),
        in_specs=[
            pl.BlockSpec((MM_BLOCK, D), lambda i: (i, 0)),
            pl.BlockSpec((D, D), lambda i: (0, 0)),
            pl.BlockSpec((1, D), lambda i: (0, 0)),
        ],
        out_specs=pl.BlockSpec((MM_BLOCK, DW), lambda i: (i, 0)),
        out_shape=jax.ShapeDtypeStruct((N, DW), jnp.int32),
    )(feats, W, b_row)


def _agg_body(x_hbm, edge_hbm, out_hbm, idx_v, xs, rows0, rows1, out0, out1,
              sem0, sem1, osem0, osem1):
    sid = lax.axis_index("s")
    wid = sid * 2 + lax.axis_index("c")

    eoff = pl.multiple_of(wid * E_PER_W, 8)

    @pl.when(wid < NW - 1)
    def _():
        pltpu.sync_copy(edge_hbm.at[pl.ds(eoff, E_PER_W)], idx_v)

    @pl.when(wid == NW - 1)
    def _():
        pltpu.sync_copy(
            edge_hbm.at[pl.ds(eoff, E_TAIL)], idx_v.at[pl.ds(0, E_TAIL)]
        )

    rps = 624
    off = pl.multiple_of(sid * rps, 8)
    pltpu.sync_copy(x_hbm.at[pl.ds(off, rps)], xs.at[pl.ds(off, rps)])

    @pl.when(sid == 0)
    def _():
        pltpu.sync_copy(
            x_hbm.at[pl.ds(16 * rps, N - 16 * rps)],
            xs.at[pl.ds(16 * rps, N - 16 * rps)],
        )

    plsc.subcore_barrier()

    bufs = (rows0, rows1)
    sems = (sem0, sem1)
    obufs = (out0, out1)
    osems = (osem0, osem1)

    def gather(g, rbuf, sem):
        return pltpu.async_copy(
            xs.at[idx_v.at[pl.ds(g * IPC, IPC)]], rbuf, sem
        )

    gather(0, rows0, sem0)
    gather(1, rows1, sem1)

    def accum(rbuf, ob):
        @plsc.parallel_loop(0, CHUNK, 1, unroll=4)
        def node_body(n):
            base = n * DEG
            for k in range(DW // LANES):
                sl = pl.ds(k * LANES, LANES)
                words = [rbuf[base + j, sl] for j in range(DEG)]
                lo = [lax.bitcast_convert_type(lax.shift_left(w, 16),
                                               jnp.float32)
                      for w in words]
                hi = [lax.bitcast_convert_type(w, jnp.float32)
                      for w in words]
                while len(lo) > 1:
                    lo = [lo[2 * i] + lo[2 * i + 1]
                          for i in range(len(lo) // 2)]
                    hi = [hi[2 * i] + hi[2 * i + 1]
                          for i in range(len(hi) // 2)]
                ob[n, sl] = lo[0]
                ob[n, pl.ds(DW + k * LANES, LANES)] = hi[0]

    def valid(g):
        return wid * NPW + g * CHUNK + CHUNK <= N

    def pair_body(p, carry):
        for b in range(2):
            g = p * 2 + b
            rbuf, sem = bufs[b], sems[b]
            ob, osem = obufs[b], osems[b]
            row0 = wid * NPW + g * CHUNK

            @pl.when(jnp.logical_and(g >= 2, valid(g - 2)))
            def _():
                pltpu.make_async_copy(
                    ob,
                    out_hbm.at[pl.ds(row0 - 2 * CHUNK, CHUNK)],
                    osem,
                ).wait()

            @pl.when(valid(g))
            def _():
                pltpu.make_async_copy(
                    xs.at[idx_v.at[pl.ds(g * IPC, IPC)]], rbuf, sem
                ).wait()
                accum(rbuf, ob)
                pltpu.async_copy(ob, out_hbm.at[pl.ds(row0, CHUNK)], osem)

            @pl.when(jnp.logical_and(g + 2 < NCHUNKS, valid(g + 2)))
            def _():
                gather(g + 2, rbuf, sem)

        return carry

    lax.fori_loop(0, NCHUNKS // 2, pair_body, 0)

    for b, last_g in ((0, NCHUNKS - 2), (1, NCHUNKS - 1)):

        @pl.when(valid(last_g))
        def _():
            pltpu.make_async_copy(
                obufs[b],
                out_hbm.at[pl.ds(wid * NPW + last_g * CHUNK, CHUNK)],
                osems[b],
            ).wait()


def _aggregate(x_words, edge_flat):
    mesh = plsc.VectorSubcoreMesh(core_axis_name="c", subcore_axis_name="s")
    agg = functools.partial(
        pl.kernel,
        out_type=jax.ShapeDtypeStruct((N, D), jnp.float32),
        mesh=mesh,
        scratch_types=[
            pltpu.VMEM((NCHUNKS * IPC,), jnp.int32),
            pltpu.VMEM_SHARED((N, DW), jnp.int32),
            pltpu.VMEM((IPC, DW), jnp.int32),
            pltpu.VMEM((IPC, DW), jnp.int32),
            pltpu.VMEM((CHUNK, D), jnp.float32),
            pltpu.VMEM((CHUNK, D), jnp.float32),
            pltpu.SemaphoreType.DMA,
            pltpu.SemaphoreType.DMA,
            pltpu.SemaphoreType.DMA,
            pltpu.SemaphoreType.DMA,
        ],
    )(_agg_body)
    return agg(x_words, edge_flat)


def kernel(feats, edge_dict, W, b):
    x_words = _linear_relu_packed(feats, W, b.reshape(1, D))
    return _aggregate(x_words, edge_dict.reshape(-1))


# speedup vs baseline: 1.0479x; 1.0479x over previous
import functools

import jax
import jax.numpy as jnp
from jax import lax
from jax.experimental import pallas as pl
from jax.experimental.pallas import tpu as pltpu
from jax.experimental.pallas import tpu_sc as plsc

N = 10000
DEG = 16
D = 256
DW = D // 2
LANES = 16

NW = 32
NPW = 320
CHUNK = 8
NCHUNKS = NPW // CHUNK
IPC = CHUNK * DEG
E_PER_W = NPW * DEG
E_TAIL = (N - (NW - 1) * NPW) * DEG

MM_BLOCK = 1000

_RND = 0x8000
_HI = -65536


def _mm_body(f_ref, w_ref, b_ref, o_ref):
    acc = lax.dot_general(
        f_ref[...], w_ref[...],
        dimension_numbers=(((1,), (1,)), ((), ())),
        preferred_element_type=jnp.float32,
    )
    r = jnp.maximum(acc + b_ref[...], 0.0) * (1.0 / DEG)
    lo = lax.bitcast_convert_type(r[:, :DW], jnp.int32)
    hi = lax.bitcast_convert_type(r[:, DW:], jnp.int32)
    o_ref[...] = lax.shift_right_logical(lo + _RND, 16) | ((hi + _RND) & _HI)


def _linear_relu_packed(feats, W, b_row):
    return pl.pallas_call(
        _mm_body,
        grid=(N // MM_BLOCK,),
        in_specs=[
            pl.BlockSpec((MM_BLOCK, D), lambda i: (i, 0)),
            pl.BlockSpec((D, D), lambda i: (0, 0)),
            pl.BlockSpec((1, D), lambda i: (0, 0)),
        ],
        out_specs=pl.BlockSpec((MM_BLOCK, DW), lambda i: (i, 0)),
        out_shape=jax.ShapeDtypeStruct((N, DW), jnp.int32),
    )(feats, W, b_row)


def _agg_body(x_hbm, edge_hbm, out_hbm, idx_v, xs, rows0, rows1, out0, out1,
              sem0, sem1, osem0, osem1):
    sid = lax.axis_index("s")
    wid = sid * 2 + lax.axis_index("c")

    eoff = pl.multiple_of(wid * E_PER_W, 8)

    @pl.when(wid < NW - 1)
    def _():
        pltpu.sync_copy(edge_hbm.at[pl.ds(eoff, E_PER_W)], idx_v)

    @pl.when(wid == NW - 1)
    def _():
        pltpu.sync_copy(
            edge_hbm.at[pl.ds(eoff, E_TAIL)], idx_v.at[pl.ds(0, E_TAIL)]
        )

    rps = 624
    off = pl.multiple_of(sid * rps, 8)
    pltpu.sync_copy(x_hbm.at[pl.ds(off, rps)], xs.at[pl.ds(off, rps)])

    @pl.when(sid == 0)
    def _():
        pltpu.sync_copy(
            x_hbm.at[pl.ds(16 * rps, N - 16 * rps)],
            xs.at[pl.ds(16 * rps, N - 16 * rps)],
        )

    plsc.subcore_barrier()

    bufs = (rows0, rows1)
    sems = (sem0, sem1)
    obufs = (out0, out1)
    osems = (osem0, osem1)

    def gather(g, rbuf, sem):
        pltpu.async_copy(
            xs.at[idx_v.at[pl.ds(g * IPC, IPC)]], rbuf, sem
        )

    gather(0, rows0, sem0)
    gather(1, rows1, sem1)

    def accum(rbuf, ob):
        @plsc.parallel_loop(0, CHUNK, 1, unroll=2)
        def node_body(n):
            base = n * DEG
            for k in range(DW // LANES):
                sl = pl.ds(k * LANES, LANES)
                words = [rbuf[base + j, sl] for j in range(DEG)]
                lo = [lax.bitcast_convert_type(lax.shift_left(w, 16),
                                               jnp.float32)
                      for w in words]
                hi = [lax.bitcast_convert_type(w, jnp.float32)
                      for w in words]
                while len(lo) > 1:
                    lo = [lo[2 * i] + lo[2 * i + 1]
                          for i in range(len(lo) // 2)]
                    hi = [hi[2 * i] + hi[2 * i + 1]
                          for i in range(len(hi) // 2)]
                ob[n, sl] = lo[0]
                ob[n, pl.ds(DW + k * LANES, LANES)] = hi[0]

    def valid(g):
        return wid * NPW + g * CHUNK + CHUNK <= N

    def pair_body(p, carry):
        for b in range(2):
            g = p * 2 + b
            rbuf, sem = bufs[b], sems[b]
            ob, osem = obufs[b], osems[b]
            row0 = wid * NPW + g * CHUNK

            @pl.when(jnp.logical_and(g >= 2, valid(g - 2)))
            def _():
                pltpu.make_async_copy(
                    ob,
                    out_hbm.at[pl.ds(row0 - 2 * CHUNK, CHUNK)],
                    osem,
                ).wait()

            @pl.when(valid(g))
            def _():
                pltpu.make_async_copy(
                    xs.at[idx_v.at[pl.ds(g * IPC, IPC)]], rbuf, sem
                ).wait()
                accum(rbuf, ob)
                pltpu.async_copy(ob, out_hbm.at[pl.ds(row0, CHUNK)], osem)

            @pl.when(jnp.logical_and(g + 2 < NCHUNKS, valid(g + 2)))
            def _():
                gather(g + 2, rbuf, sem)

        return carry

    lax.fori_loop(0, NCHUNKS // 2, pair_body, 0)

    for b, last_g in ((0, NCHUNKS - 2), (1, NCHUNKS - 1)):

        @pl.when(valid(last_g))
        def _():
            pltpu.make_async_copy(
                obufs[b],
                out_hbm.at[pl.ds(wid * NPW + last_g * CHUNK, CHUNK)],
                osems[b],
            ).wait()


def _aggregate(x_words, edge_flat):
    mesh = plsc.VectorSubcoreMesh(core_axis_name="c", subcore_axis_name="s")
    agg = functools.partial(
        pl.kernel,
        out_type=jax.ShapeDtypeStruct((N, D), jnp.float32),
        mesh=mesh,
        scratch_types=[
            pltpu.VMEM((NCHUNKS * IPC,), jnp.int32),
            pltpu.VMEM_SHARED((N, DW), jnp.int32),
            pltpu.VMEM((CHUNK * DEG, DW), jnp.int32),
            pltpu.VMEM((CHUNK * DEG, DW), jnp.int32),
            pltpu.VMEM((CHUNK, D), jnp.float32),
            pltpu.VMEM((CHUNK, D), jnp.float32),
            pltpu.SemaphoreType.DMA,
            pltpu.SemaphoreType.DMA,
            pltpu.SemaphoreType.DMA,
            pltpu.SemaphoreType.DMA,
        ],
    )(_agg_body)
    return agg(x_words, edge_flat)


def kernel(feats, edge_dict, W, b):
    x_words = _linear_relu_packed(feats, W, b.reshape(1, D))
    return _aggregate(x_words, edge_dict.reshape(-1))
